# baseline (speedup 1.0000x reference)
import functools

import jax
import jax.numpy as jnp
import numpy as np
from jax import lax
from jax.experimental import pallas as pl
from jax.experimental.pallas import tpu as pltpu
from jax.experimental.pallas import tpu_sc as plsc

N = 10000
E = 640000
DIN = 128
HID = 512
LAT = 30
NEG = 0.2

LPAD = 32
NPAD = 10112
ROWS_PER_TILE = 160
E_PAD = 32 * ROWS_PER_TILE * 128
OUT_ROWS_PER_TILE = NPAD // 16

CROWS = 2
NCHUNKS = ROWS_PER_TILE // CROWS
NBUF = 4

_QPERM = np.empty(LPAD, dtype=np.int32)
_QPERM[0::2] = np.arange(16)
_QPERM[1::2] = np.arange(16) + 16
_DENOM_POS = int(np.where(_QPERM == 30)[0][0])


def _elu(v):
    return jnp.where(v > 0, v, jnp.exp(jnp.minimum(v, 0.0)) - 1.0)


def _enc_body(x_ref, w1_ref, b1_ref, gw_ref, gwq_ref, atti_ref, attj_ref,
              xw_ref, a_ref, xwq_ref):
    h = jnp.dot(x_ref[...], w1_ref[...], preferred_element_type=jnp.float32)
    h = _elu(h + b1_ref[...])
    xw = jnp.dot(h, gw_ref[...], preferred_element_type=jnp.float32)
    aD = jnp.sum(xw * atti_ref[...], axis=1, keepdims=True)
    aS = jnp.sum(xw * attj_ref[...], axis=1, keepdims=True)
    lane = lax.broadcasted_iota(jnp.int32, xw.shape, 1)
    xw_ref[...] = xw + jnp.where(lane == 30, 1.0, 0.0)
    a_ref[...] = jnp.concatenate([aD, aS], axis=1)
    xwq = jnp.dot(h, gwq_ref[...], preferred_element_type=jnp.float32)
    xwq = xwq + jnp.where(lane == _DENOM_POS, 1.0, 0.0)
    xwq_ref[...] = xwq.astype(jnp.bfloat16)


def _dec_body(p0_ref, p1_ref, xw_ref, a_ref, bias_ref, dw_ref, db1_ref,
              w1_ref, db2_ref, out_ref):
    a = a_ref[...]
    s = a[:, 0:1] + a[:, 1:2]
    s = jnp.where(s >= 0.0, s, NEG * s)
    wself = jnp.exp(s)
    num = p0_ref[...] + p1_ref[...] + wself * xw_ref[...]
    lane = lax.broadcasted_iota(jnp.int32, num.shape, 1)
    m30 = jnp.where(lane == 30, 1.0, 0.0)
    denom = jnp.sum(num * m30, axis=1, keepdims=True) + 1e-16
    z = _elu(num / denom + bias_ref[...])
    hd = jnp.dot(z, dw_ref[...], preferred_element_type=jnp.float32)
    hd = _elu(hd + db1_ref[...])
    out = lax.dot_general(hd, w1_ref[...], (((1,), (1,)), ((), ())),
                          preferred_element_type=jnp.float32)
    out_ref[...] = out + db2_ref[...]


def _vbcast(v, e):
    idx = jnp.full((16, 1), e, dtype=jnp.int32)
    dn = lax.GatherDimensionNumbers(offset_dims=(), collapsed_slice_dims=(0,),
                                    start_index_map=(0,))
    return lax.gather(v, idx, dn, (1,),
                      mode=lax.GatherScatterMode.PROMISE_IN_BOUNDS)


def _edge_body(src_hbm, dst_hbm, ad_hbm, as_hbm, xw_hbm, zero_hbm, out_hbm,
               adv, asv, idx_s, idx_d, rows0, rows1, rows2, rows3,
               fbuf0, fbuf1, acc, g0, g1, g2, g3, s0, s1):
    cid = lax.axis_index("c")
    sid = lax.axis_index("s")
    wid = sid * 2 + cid
    rows = [rows0, rows1, rows2, rows3]
    fbuf = [fbuf0, fbuf1]
    gsem = [g0, g1, g2, g3]
    ssem = [s0, s1]

    pltpu.sync_copy(zero_hbm, acc.at[pl.ds(sid * OUT_ROWS_PER_TILE, OUT_ROWS_PER_TILE)])
    pltpu.sync_copy(ad_hbm, adv)
    pltpu.sync_copy(as_hbm, asv)
    pltpu.sync_copy(src_hbm.at[pl.ds(wid * ROWS_PER_TILE, ROWS_PER_TILE)], idx_s)
    pltpu.sync_copy(dst_hbm.at[pl.ds(wid * ROWS_PER_TILE, ROWS_PER_TILE)], idx_d)
    plsc.subcore_barrier()

    def stage_a(c, b):
        @pl.when(c < NCHUNKS)
        def _():
            for j in range(CROWS):
                pltpu.async_copy(xw_hbm.at[idx_s.at[c * CROWS + j]],
                                 rows[b].at[pl.ds(j * 128, 128)], gsem[b])

    def drain_scatters(p):
        for j in range(CROWS):
            pltpu.make_async_copy(fbuf[p].at[pl.ds(j * 128, 128)],
                                  acc.at[idx_d.at[0]], ssem[p]).wait()

    def stage_b(c, b):
        p = b % 2
        for j in range(CROWS):
            pltpu.make_async_copy(xw_hbm.at[idx_s.at[0]],
                                  rows[b].at[pl.ds(j * 128, 128)], gsem[b]).wait()

        @pl.when(c >= 2)
        def _():
            drain_scatters(p)

        def grp_body(g, carry2):
            row = c * CROWS + g // 8
            lane0 = (g % 8) * 16
            srci = idx_s[row, pl.ds(lane0, 16)]
            dsti = idx_d[row, pl.ds(lane0, 16)]
            av = plsc.load_gather(adv, [dsti])
            bv = plsc.load_gather(asv, [srci])
            s = av + bv
            s = jnp.where(s >= 0.0, s, NEG * s)
            w = jnp.exp(s)
            ebase = g * 16
            for e in range(16):
                bw = _vbcast(w, e)
                vi = plsc.bitcast(rows[b][ebase + e, :], jnp.int32)
                even = plsc.bitcast(vi << 16, jnp.float32)
                odd = plsc.bitcast(vi & jnp.int32(-65536), jnp.float32)
                fbuf[p][ebase + e, pl.ds(0, 16)] = even * bw
                fbuf[p][ebase + e, pl.ds(16, 16)] = odd * bw
            return carry2

        lax.fori_loop(0, CROWS * 8, grp_body, 0)

        for j in range(CROWS):
            pltpu.async_copy(fbuf[p].at[pl.ds(j * 128, 128)],
                             acc.at[idx_d.at[c * CROWS + j]], ssem[p], add=True)

    stage_a(0, 0)
    stage_a(1, 1)

    def super_body(k, carry):
        c = k * 4
        stage_a(c + 2, 2)
        stage_b(c + 0, 0)
        stage_a(c + 3, 3)
        stage_b(c + 1, 1)
        stage_a(c + 4, 0)
        stage_b(c + 2, 2)
        stage_a(c + 5, 1)
        stage_b(c + 3, 3)
        return carry

    lax.fori_loop(0, NCHUNKS // 4, super_body, 0)
    for p in range(2):
        drain_scatters(p)
    plsc.subcore_barrier()
    pltpu.sync_copy(acc.at[pl.ds(sid * OUT_ROWS_PER_TILE, OUT_ROWS_PER_TILE)],
                    out_hbm.at[cid, pl.ds(sid * OUT_ROWS_PER_TILE, OUT_ROWS_PER_TILE)])


@functools.cache
def _edge_call():
    return pl.kernel(
        _edge_body,
        out_type=jax.ShapeDtypeStruct((2, NPAD, LPAD), jnp.float32),
        mesh=plsc.VectorSubcoreMesh(core_axis_name="c", subcore_axis_name="s"),
        scratch_types=(
            [
                pltpu.VMEM((NPAD,), jnp.float32),
                pltpu.VMEM((NPAD,), jnp.float32),
                pltpu.VMEM((ROWS_PER_TILE, 128), jnp.int32),
                pltpu.VMEM((ROWS_PER_TILE, 128), jnp.int32),
            ]
            + [pltpu.VMEM((CROWS * 128, LPAD), jnp.bfloat16) for _ in range(NBUF)]
            + [pltpu.VMEM((CROWS * 128, LPAD), jnp.float32) for _ in range(2)]
            + [pltpu.VMEM_SHARED((NPAD, LPAD), jnp.float32)]
            + [pltpu.SemaphoreType.DMA for _ in range(NBUF + 2)]
        ),
        compiler_params=pltpu.CompilerParams(needs_layout_passes=False,
                                             use_tc_tiling_on_sc=False),
    )


def kernel(x, edge_index, W1, b1, gat_weight, gat_att, gat_bias, dec_W1,
           dec_b1, dec_b2):
    f32 = jnp.float32
    attv = gat_att.reshape(2 * LAT)
    atti = jnp.pad(attv[:LAT], (0, LPAD - LAT)).reshape(1, LPAD)
    attj = jnp.pad(attv[LAT:], (0, LPAD - LAT)).reshape(1, LPAD)
    gw_p = jnp.pad(gat_weight, ((0, 0), (0, LPAD - LAT)))
    gw_q = gw_p[:, _QPERM]

    M = 2000
    grid = (N // M,)
    xwext, aDS, xwq = pl.pallas_call(
        _enc_body,
        grid=grid,
        in_specs=[
            pl.BlockSpec((M, DIN), lambda i: (i, 0)),
            pl.BlockSpec((DIN, HID), lambda i: (0, 0)),
            pl.BlockSpec((1, HID), lambda i: (0, 0)),
            pl.BlockSpec((HID, LPAD), lambda i: (0, 0)),
            pl.BlockSpec((HID, LPAD), lambda i: (0, 0)),
            pl.BlockSpec((1, LPAD), lambda i: (0, 0)),
            pl.BlockSpec((1, LPAD), lambda i: (0, 0)),
        ],
        out_specs=[
            pl.BlockSpec((M, LPAD), lambda i: (i, 0)),
            pl.BlockSpec((M, 2), lambda i: (i, 0)),
            pl.BlockSpec((M, LPAD), lambda i: (i, 0)),
        ],
        out_shape=[
            jax.ShapeDtypeStruct((N, LPAD), f32),
            jax.ShapeDtypeStruct((N, 2), f32),
            jax.ShapeDtypeStruct((N, LPAD), jnp.bfloat16),
        ],
    )(x, W1, b1.reshape(1, HID), gw_p, gw_q, atti, attj)

    pad_e = E_PAD - E
    pad_dst = jnp.arange(pad_e, dtype=jnp.int32) % NPAD
    srcp = jnp.concatenate([edge_index[0], jnp.full((pad_e,), N, jnp.int32)])
    dstp = jnp.concatenate([edge_index[1], pad_dst])
    srcp = srcp.reshape(E_PAD // 128, 128)
    dstp = dstp.reshape(E_PAD // 128, 128)
    adp = jnp.pad(aDS[:, 0], (0, NPAD - N))
    asp = jnp.pad(aDS[:, 1], (0, NPAD - N))
    xwq_p = jnp.pad(xwq, ((0, NPAD - N), (0, 0)))
    zeros = jnp.zeros((OUT_ROWS_PER_TILE, LPAD), f32)

    part = _edge_call()(srcp, dstp, adp, asp, xwq_p, zeros)

    bias_p = jnp.pad(gat_bias, (0, LPAD - LAT)).reshape(1, LPAD)
    dw_p = jnp.pad(dec_W1, ((0, LPAD - LAT), (0, 0)))
    out = pl.pallas_call(
        _dec_body,
        grid=grid,
        in_specs=[
            pl.BlockSpec((M, LPAD), lambda i: (i, 0)),
            pl.BlockSpec((M, LPAD), lambda i: (i, 0)),
            pl.BlockSpec((M, LPAD), lambda i: (i, 0)),
            pl.BlockSpec((M, 2), lambda i: (i, 0)),
            pl.BlockSpec((1, LPAD), lambda i: (0, 0)),
            pl.BlockSpec((LPAD, HID), lambda i: (0, 0)),
            pl.BlockSpec((1, HID), lambda i: (0, 0)),
            pl.BlockSpec((DIN, HID), lambda i: (0, 0)),
            pl.BlockSpec((1, DIN), lambda i: (0, 0)),
        ],
        out_specs=pl.BlockSpec((M, DIN), lambda i: (i, 0)),
        out_shape=jax.ShapeDtypeStruct((N, DIN), f32),
    )(part[0, :N], part[1, :N], xwext, aDS, bias_p, dw_p,
      dec_b1.reshape(1, HID), W1, dec_b2.reshape(1, DIN))
    return out
